# baseline (speedup 1.0000x reference)
import functools

import jax
import jax.numpy as jnp
from jax import lax
from jax.experimental import pallas as pl
from jax.experimental.pallas import tpu as pltpu
from jax.experimental.pallas import tpu_sc as plsc

N = 10000
E = 320000
H = 128
L = 4
HEADS = 8
HD = 16
B = 64

NP = 10112
NW = 32
CH = 128
EPW = 10112
NCH = EPW // CH
RPT = NP // 16

_f32 = jnp.float32


def _bcast_lane(z, j):
    idx = jnp.full((16, 1), j, jnp.int32)
    dn = lax.GatherDimensionNumbers(
        offset_dims=(), collapsed_slice_dims=(0,), start_index_map=(0,))
    return lax.gather(z, idx, dn, (1,),
                      mode=lax.GatherScatterMode.PROMISE_IN_BOUNDS)


def _exp32(x):
    y = x * 1.4426950408889634
    ki = y.astype(jnp.int32)
    f = y - ki.astype(_f32)
    g = f * 0.6931471805599453
    p = 1.0 / 40320.0
    p = p * g + 1.0 / 5040.0
    p = p * g + 1.0 / 720.0
    p = p * g + 1.0 / 120.0
    p = p * g + 1.0 / 24.0
    p = p * g + 1.0 / 6.0
    p = p * g + 0.5
    p = p * g + 1.0
    p = p * g + 1.0
    sc = lax.bitcast_convert_type((ki + 127) * 8388608, _f32)
    return p * sc


_sc_mesh = plsc.VectorSubcoreMesh(
    core_axis_name="c", subcore_axis_name="s", num_cores=2, num_subcores=16)


@functools.partial(
    pl.kernel,
    out_type=(jax.ShapeDtypeStruct((2, NP, H), _f32),
              jax.ShapeDtypeStruct((2, NP, H), _f32)),
    mesh=_sc_mesh,
    scratch_types=[
        pltpu.VMEM((CH,), jnp.int32),
        pltpu.VMEM((CH,), jnp.int32),
        pltpu.VMEM((CH, H), _f32),
        pltpu.VMEM((CH, H), _f32),
        pltpu.VMEM((CH, H), _f32),
        pltpu.VMEM_SHARED((NP, H), _f32),
        pltpu.SemaphoreType.DMA,
        pltpu.SemaphoreType.DMA,
        pltpu.SemaphoreType.DMA,
    ],
)
def _gat_edges(h_hbm, as_hbm, ad_hbm, src_hbm, dst_hbm, accm_hbm, accz_hbm,
               sidx, didx, grows, asrows, adrows, acc_sh, sem1, sem2, sem3):
    c = lax.axis_index("c")
    s = lax.axis_index("s")
    w = s * 2 + c

    def _z_of(r):
        t = asrows[r, pl.ds(0, 16)] + adrows[r, pl.ds(0, 16)]
        return _exp32(jnp.maximum(t, t * 0.2))

    def _zero_grows():
        def _zero_row(r, _):
            for j in range(H // 16):
                grows[r, pl.ds(j * 16, 16)] = jnp.zeros((16,), _f32)
            return _
        lax.fori_loop(0, CH, _zero_row, None)

    def _zero_own_slab():
        def _zero_acc(k, _):
            pltpu.sync_copy(grows, acc_sh.at[pl.ds(s * RPT + k * CH, CH)])
            return _
        lax.fori_loop(0, RPT // CH, _zero_acc, None)
        if RPT % CH:
            pltpu.sync_copy(grows.at[pl.ds(0, RPT % CH)],
                            acc_sh.at[pl.ds(s * RPT + RPT - RPT % CH,
                                            RPT % CH)])

    _zero_grows()
    _zero_own_slab()
    plsc.subcore_barrier()

    def _chunk1(ch, _):
        pltpu.sync_copy(src_hbm.at[w, ch], sidx)
        pltpu.sync_copy(dst_hbm.at[w, ch], didx)
        cp1 = pltpu.async_copy(h_hbm.at[sidx], grows, sem1)
        cp2 = pltpu.async_copy(as_hbm.at[sidx], asrows, sem2)
        cp3 = pltpu.async_copy(ad_hbm.at[didx], adrows, sem3)
        cp1.wait()
        cp2.wait()
        cp3.wait()

        def _edge(rr, _):
            for u in range(4):
                r = rr * 4 + u
                z = _z_of(r)
                for j in range(HEADS):
                    zb = _bcast_lane(z, j)
                    grows[r, pl.ds(j * 16, 16)] = (
                        grows[r, pl.ds(j * 16, 16)] * zb)
            return _
        lax.fori_loop(0, CH // 4, _edge, None)

        pltpu.sync_copy(grows, acc_sh.at[didx], add=True)
        return _
    lax.fori_loop(0, NCH, _chunk1, None)

    plsc.subcore_barrier()
    pltpu.sync_copy(acc_sh.at[pl.ds(s * RPT, RPT)],
                    accm_hbm.at[c, pl.ds(s * RPT, RPT)])

    _zero_grows()
    _zero_own_slab()
    plsc.subcore_barrier()

    def _chunk2(ch, _):
        pltpu.sync_copy(dst_hbm.at[w, ch], didx)
        pltpu.sync_copy(src_hbm.at[w, ch], sidx)
        cp2 = pltpu.async_copy(as_hbm.at[sidx], asrows, sem2)
        cp3 = pltpu.async_copy(ad_hbm.at[didx], adrows, sem3)
        cp2.wait()
        cp3.wait()

        def _edge(rr, _):
            for u in range(4):
                r = rr * 4 + u
                grows[r, pl.ds(0, 16)] = _z_of(r)
            return _
        lax.fori_loop(0, CH // 4, _edge, None)

        pltpu.sync_copy(grows, acc_sh.at[didx], add=True)
        return _
    lax.fori_loop(0, NCH, _chunk2, None)

    plsc.subcore_barrier()
    pltpu.sync_copy(acc_sh.at[pl.ds(s * RPT, RPT)],
                    accz_hbm.at[c, pl.ds(s * RPT, RPT)])


def _exp32_tc(x):
    y = x * 1.4426950408889634
    ki = y.astype(jnp.int32)
    f = y - ki.astype(_f32)
    g = f * 0.6931471805599453
    p = 1.0 / 40320.0
    p = p * g + 1.0 / 5040.0
    p = p * g + 1.0 / 720.0
    p = p * g + 1.0 / 120.0
    p = p * g + 1.0 / 24.0
    p = p * g + 1.0 / 6.0
    p = p * g + 0.5
    p = p * g + 1.0
    p = p * g + 1.0
    sc = lax.bitcast_convert_type((ki + 127) * 8388608, _f32)
    return p * sc


def _head_expand_mat():
    r = lax.broadcasted_iota(jnp.int32, (HEADS, H), 0)
    c = lax.broadcasted_iota(jnp.int32, (HEADS, H), 1)
    return (r == c // HD).astype(_f32)


def _tc_call(body, out_shapes, *args):
    return pl.pallas_call(body, out_shape=out_shapes)(*args)


def _emit_layer_inputs(y, w_ref, as_ref, ad_ref, h_ref, asout_ref, adout_ref,
                       zsx_ref):
    h = jnp.dot(y, w_ref[...], preferred_element_type=_f32, precision=lax.Precision.HIGHEST)
    st = _head_expand_mat()
    a_s = jnp.dot(h * as_ref[...], st.T, preferred_element_type=_f32, precision=lax.Precision.HIGHEST)
    a_d = jnp.dot(h * ad_ref[...], st.T, preferred_element_type=_f32, precision=lax.Precision.HIGHEST)
    rowpad = jnp.zeros((NP - N, H), _f32)
    h_ref[...] = jnp.concatenate([h, rowpad], axis=0)
    zpad120 = jnp.zeros((N, H - HEADS), _f32)
    asout_ref[...] = jnp.concatenate(
        [jnp.concatenate([a_s, zpad120], axis=1), rowpad], axis=0)
    adout_ref[...] = jnp.concatenate(
        [jnp.concatenate([a_d, zpad120], axis=1), rowpad], axis=0)
    t = a_s + a_d
    z_self = _exp32_tc(jnp.maximum(t, t * 0.2))
    zsx = jnp.dot(z_self, st, preferred_element_type=_f32,
                  precision=lax.Precision.HIGHEST)
    zsx_ref[...] = jnp.concatenate([zsx, rowpad], axis=0)


def _prologue_body(x_ref, wp_ref, bp_ref, w_ref, as_ref, ad_ref,
                   h_ref, asout_ref, adout_ref, zsx_ref):
    x = x_ref[...]
    y = jax.nn.relu(jnp.dot(x, wp_ref[...],
                            preferred_element_type=_f32, precision=lax.Precision.HIGHEST) + bp_ref[...])
    _emit_layer_inputs(y, w_ref, as_ref, ad_ref, h_ref, asout_ref, adout_ref,
                       zsx_ref)


def _mid_body(xn_ref, w_ref, as_ref, ad_ref, h_ref, asout_ref, adout_ref,
              zsx_ref):
    _emit_layer_inputs(xn_ref[...], w_ref, as_ref, ad_ref,
                       h_ref, asout_ref, adout_ref, zsx_ref)


def _combine_body(a0_ref, a1_ref, z0_ref, z1_ref, am_ref, az_ref):
    am_ref[...] = a0_ref[...] + a1_ref[...]
    az_ref[...] = z0_ref[...] + z1_ref[...]


def _post_body(accm_ref, accz_ref, h_ref, zsx_ref,
               b_ref, gam_ref, bet_ref, xn_ref):
    h = h_ref[...][:N, :]
    zsx = zsx_ref[...][:N, :]
    accm = accm_ref[...][:N, :]
    accz = accz_ref[...][:N, :HEADS]
    st = _head_expand_mat()
    dx = jnp.dot(accz, st, preferred_element_type=_f32, precision=lax.Precision.HIGHEST) + zsx
    out = (accm + h * zsx) / dx + b_ref[...]
    mu = jnp.mean(out, axis=0, keepdims=True)
    var = jnp.mean((out - mu) ** 2, axis=0, keepdims=True)
    v = var + 1e-5
    r = lax.rsqrt(v)
    r = r * (1.5 - 0.5 * v * r * r)
    xn_ref[...] = jax.nn.relu(
        gam_ref[...] * (out - mu) * r + bet_ref[...])


def _head_body(y_ref, batch_ref, wv_ref, bv_ref, wo_ref, bo_ref,
               fw1_ref, fw2_ref, fb_ref, pw1_ref, pb1_ref, pw2_ref, pb2_ref,
               out_ref):
    y = y_ref[...]
    bt = batch_ref[...]
    gid = lax.broadcasted_iota(jnp.int32, (1, B), 1)
    oh = (bt == gid).astype(_f32)
    cnt = jnp.sum(oh, axis=0, keepdims=True)
    ysum = lax.dot_general(oh, y, (((0,), (0,)), ((), ())),
                           preferred_element_type=_f32, precision=lax.Precision.HIGHEST)
    ys = ysum / jnp.maximum(cnt, 1.0).T
    v = jnp.dot(ys, wv_ref[...], preferred_element_type=_f32, precision=lax.Precision.HIGHEST) + bv_ref[...]
    attn = jnp.dot(v, wo_ref[...], preferred_element_type=_f32, precision=lax.Precision.HIGHEST) + bo_ref[...]
    fz = jax.nn.relu(
        jnp.dot(attn, fw1_ref[...], preferred_element_type=_f32, precision=lax.Precision.HIGHEST)
        + jnp.dot(ys, fw2_ref[...], preferred_element_type=_f32, precision=lax.Precision.HIGHEST)
        + fb_ref[...])
    h1 = jax.nn.relu(
        jnp.dot(fz, pw1_ref[...], preferred_element_type=_f32, precision=lax.Precision.HIGHEST) + pb1_ref[...])
    out_ref[...] = jnp.dot(h1, pw2_ref[...],
                           preferred_element_type=_f32, precision=lax.Precision.HIGHEST) + pb2_ref[...]


def kernel(seq_x, seq_edge_index, seq_batch, struct_x, struct_edge_index,
           struct_batch, seq_proj_W, seq_proj_b, gcn_W, gcn_b, seq_gamma,
           seq_beta, struct_proj_W, struct_proj_b, gat_W, gat_att_src,
           gat_att_dst, gat_b, struct_gamma, struct_beta, Wq, bq, Wk, bk,
           Wv, bv, Wo, bo, fusion_W, fusion_b, pred_W1, pred_b1, pred_W2,
           pred_b2):
    srcp = jnp.pad(struct_edge_index[0].reshape(NW, E // NW),
                   ((0, 0), (0, EPW - E // NW)),
                   constant_values=N).reshape(NW, NCH, CH)
    dstp = jnp.pad(struct_edge_index[1].reshape(NW, E // NW),
                   ((0, 0), (0, EPW - E // NW)),
                   constant_values=N).reshape(NW, NCH, CH)

    hsd = jax.ShapeDtypeStruct((NP, H), _f32)
    sdsd = jax.ShapeDtypeStruct((NP, H), _f32)
    xnsd = jax.ShapeDtypeStruct((N, H), _f32)

    h, a_s, a_d, zsx = _tc_call(
        _prologue_body, (hsd, sdsd, sdsd, hsd),
        struct_x, struct_proj_W, struct_proj_b.reshape(1, H),
        gat_W[0], gat_att_src[0].reshape(1, H), gat_att_dst[0].reshape(1, H))

    for i in range(L):
        accm, accz = _gat_edges(h, a_s, a_d, srcp, dstp)
        accmc, acczc = _tc_call(
            _combine_body, (hsd, hsd),
            accm[0], accm[1], accz[0], accz[1])
        xn = _tc_call(
            _post_body, xnsd,
            accmc, acczc, h, zsx,
            gat_b[i].reshape(1, H),
            struct_gamma[i].reshape(1, H), struct_beta[i].reshape(1, H))
        if i + 1 < L:
            h, a_s, a_d, zsx = _tc_call(
                _mid_body, (hsd, sdsd, sdsd, hsd),
                xn, gat_W[i + 1], gat_att_src[i + 1].reshape(1, H),
                gat_att_dst[i + 1].reshape(1, H))

    return _tc_call(
        _head_body, jax.ShapeDtypeStruct((B, 1), _f32),
        xn, struct_batch.reshape(N, 1), Wv, bv.reshape(1, H), Wo,
        bo.reshape(1, H), fusion_W[:H], fusion_W[H:],
        fusion_b.reshape(1, H), pred_W1, pred_b1.reshape(1, H // 2),
        pred_W2, pred_b2.reshape(1, 1))
